# baseline (speedup 1.0000x reference)
import functools

import jax
import jax.numpy as jnp
from jax.experimental import pallas as pl

_KS = (2000, 1000, 500)


def _blk8(dim, pref):
    cands = [b for b in range(8, min(dim, pref) + 1, 8) if dim % b == 0]
    return max(cands) if cands else dim


def _blk128(dim, pref):
    cands = [b for b in range(128, min(dim, pref) + 1, 128) if dim % b == 0]
    return max(cands) if cands else dim


def _mm_kernel(a_ref, b_ref, o_ref, *, nc, zero_diag, bi, bj):
    c = pl.program_id(2)

    @pl.when(c == 0)
    def _():
        o_ref[...] = jnp.zeros_like(o_ref)

    o_ref[...] += jnp.dot(a_ref[...], b_ref[...],
                          preferred_element_type=jnp.float32)

    if zero_diag:
        i = pl.program_id(0)
        j = pl.program_id(1)

        @pl.when(c == nc - 1)
        def _():
            gi = i * bi + jax.lax.broadcasted_iota(jnp.int32, (bi, bj), 0)
            gj = j * bj + jax.lax.broadcasted_iota(jnp.int32, (bi, bj), 1)
            o_ref[...] = jnp.where(gi == gj, 0.0, o_ref[...])


def mm(a, b, zero_diag=False):
    m, k = a.shape
    k2, n = b.shape
    assert k == k2
    bi = _blk8(m, 512)
    bj = _blk128(n, 2048)
    bc = _blk128(k, 2048)
    grid = (m // bi, n // bj, k // bc)
    return pl.pallas_call(
        functools.partial(_mm_kernel, nc=grid[2], zero_diag=zero_diag,
                          bi=bi, bj=bj),
        grid=grid,
        in_specs=[
            pl.BlockSpec((bi, bc), lambda i, j, c: (i, c)),
            pl.BlockSpec((bc, bj), lambda i, j, c: (c, j)),
        ],
        out_specs=pl.BlockSpec((bi, bj), lambda i, j, c: (i, j)),
        out_shape=jax.ShapeDtypeStruct((m, n), jnp.float32),
    )(a, b)


def _mmt_kernel(a_ref, h_ref, o_ref):
    c = pl.program_id(1)

    @pl.when(c == 0)
    def _():
        o_ref[...] = jnp.zeros_like(o_ref)

    o_ref[...] += jax.lax.dot_general(
        a_ref[...].astype(jnp.float32), h_ref[...], (((0,), (0,)), ((), ())),
        preferred_element_type=jnp.float32)


def mmt(a, h):
    s, d = a.shape
    s2, f = h.shape
    assert s == s2
    bd = _blk128(d, 1024)
    bs = _blk8(s, 2048)
    grid = (d // bd, s // bs)
    return pl.pallas_call(
        _mmt_kernel,
        grid=grid,
        in_specs=[
            pl.BlockSpec((bs, bd), lambda i, c: (c, i)),
            pl.BlockSpec((bs, f), lambda i, c: (c, 0)),
        ],
        out_specs=pl.BlockSpec((bd, f), lambda i, c: (i, 0)),
        out_shape=jax.ShapeDtypeStruct((d, f), jnp.float32),
    )(a, h)


def _scale_mm_kernel(x_ref, w_ref, s_ref, o_ref):
    o_ref[...] = s_ref[...] * jnp.dot(x_ref[...], w_ref[...],
                                      preferred_element_type=jnp.float32)


def scale_mm(x, w, s):
    n, f = x.shape
    bn = _blk8(n, 2048)
    grid = (n // bn,)
    return pl.pallas_call(
        _scale_mm_kernel,
        grid=grid,
        in_specs=[
            pl.BlockSpec((bn, f), lambda i: (i, 0)),
            pl.BlockSpec((f, w.shape[1]), lambda i: (0, 0)),
            pl.BlockSpec((bn, 1), lambda i: (i, 0)),
        ],
        out_specs=pl.BlockSpec((bn, w.shape[1]), lambda i: (i, 0)),
        out_shape=jax.ShapeDtypeStruct((n, w.shape[1]), jnp.float32),
    )(x, w, s)


def _epilogue_kernel(z_ref, hs_ref, s_ref, b_ref, o_ref, *, relu):
    o = s_ref[...] * (z_ref[...] + 2.0 * hs_ref[...]) + b_ref[...]
    if relu:
        o = jnp.maximum(o, 0.0)
    o_ref[...] = o


def gcn_epilogue(z, hs, s, b, relu):
    n, f = z.shape
    bn = _blk8(n, 2048)
    grid = (n // bn,)
    return pl.pallas_call(
        functools.partial(_epilogue_kernel, relu=relu),
        grid=grid,
        in_specs=[
            pl.BlockSpec((bn, f), lambda i: (i, 0)),
            pl.BlockSpec((bn, f), lambda i: (i, 0)),
            pl.BlockSpec((bn, 1), lambda i: (i, 0)),
            pl.BlockSpec((1, f), lambda i: (0, 0)),
        ],
        out_specs=pl.BlockSpec((bn, f), lambda i: (i, 0)),
        out_shape=jax.ShapeDtypeStruct((n, f), jnp.float32),
    )(z, hs, s, b.reshape(1, f))


def _dense_gcn_kernel(a_ref, x_ref, w_ref, b_ref, o_ref, *, relu):
    a = a_ref[...]
    h = jnp.dot(x_ref[...], w_ref[...], preferred_element_type=jnp.float32)
    deg = jnp.sum(a, axis=0) + 2.0
    dinv = jax.lax.rsqrt(deg)
    hs = dinv[:, None] * h
    z = jax.lax.dot_general(a, hs, (((0,), (0,)), ((), ())),
                            preferred_element_type=jnp.float32)
    o = dinv[:, None] * (z + 2.0 * hs) + b_ref[...]
    if relu:
        o = jnp.maximum(o, 0.0)
    o_ref[...] = o


def dense_gcn(a, x, w, b, relu):
    k, f = x.shape
    return pl.pallas_call(
        functools.partial(_dense_gcn_kernel, relu=relu),
        in_specs=[
            pl.BlockSpec(a.shape, lambda: (0, 0)),
            pl.BlockSpec((k, f), lambda: (0, 0)),
            pl.BlockSpec(w.shape, lambda: (0, 0)),
            pl.BlockSpec((1, w.shape[1]), lambda: (0, 0)),
        ],
        out_specs=pl.BlockSpec((k, w.shape[1]), lambda: (0, 0)),
        out_shape=jax.ShapeDtypeStruct((k, w.shape[1]), jnp.float32),
    )(a, x, w, b.reshape(1, -1))


def _logsoftmax_kernel(x_ref, o_ref):
    x = x_ref[...]
    m = jnp.max(x, axis=1, keepdims=True)
    lse = m + jnp.log(jnp.sum(jnp.exp(x - m), axis=1, keepdims=True))
    o_ref[...] = x - lse


def row_logsoftmax(x):
    n, f = x.shape
    bn = _blk8(n, 2048)
    return pl.pallas_call(
        _logsoftmax_kernel,
        grid=(n // bn,),
        in_specs=[pl.BlockSpec((bn, f), lambda i: (i, 0))],
        out_specs=pl.BlockSpec((bn, f), lambda i: (i, 0)),
        out_shape=jax.ShapeDtypeStruct((n, f), jnp.float32),
    )(x)


def kernel(x, edge_index, dW0, dW1, dW2, dW3, db0, db1, db2, db3,
           pw0, pw1, pw2, uW0, uW1, uW2, ub0, ub1, ub2):
    n = x.shape[0]
    np_ = ((n + 127) // 128) * 128
    if np_ % 1024:
        np_ = ((n + 1023) // 1024) * 1024
    adj = jnp.zeros((np_, np_), jnp.bfloat16).at[
        edge_index[0], edge_index[1]].add(1.0)

    xpad = jnp.zeros((np_, x.shape[1]), jnp.float32).at[:n].set(x)

    deg0 = jnp.sum(adj, axis=0, dtype=jnp.float32) + 2.0
    dinv0 = jax.lax.rsqrt(deg0).reshape(np_, 1)
    hs = scale_mm(xpad, dW0, dinv0)
    z = mmt(adj, hs)
    x1 = gcn_epilogue(z, hs, dinv0, db0, relu=True)[:n]

    pws = (pw0, pw1, pw2)
    dWs = (dW1, dW2, dW3)
    dbs = (db1, db2, db3)

    xs = [x1]
    adjs = [None]
    perms = []

    cur_x = x1
    cur_src = adj
    for lvl in range(3):
        k = _KS[lvl]
        w = pws[lvl]
        score = cur_x @ (w / jnp.linalg.norm(w))
        top_vals, perm = jax.lax.top_k(score, k)
        xp = cur_x[perm] * jnp.tanh(top_vals)[:, None]
        kk = jnp.arange(k)
        one = jnp.ones((), cur_src.dtype)
        if lvl == 0:
            src, dst = edge_index[0], edge_index[1]
            big = jnp.int32(1 << 30)
            inv = jnp.full((n,), big, jnp.int32).at[perm].set(kk.astype(jnp.int32))
            keep = src != dst
            rs = jnp.where(keep, inv[src], big)
            rd = jnp.where(keep, inv[dst], big)
            g1 = (jnp.zeros((k, np_), cur_src.dtype).at[kk, perm].set(one)
                  .at[rs, dst].add(one, mode='drop'))
            g2 = (jnp.zeros((np_, k), cur_src.dtype).at[perm, kk].set(one)
                  .at[src, rd].add(one, mode='drop'))
        else:
            g1 = cur_src[perm, :].at[kk, perm].set(one)
            g2 = cur_src[:, perm].at[perm, kk].set(one)
        p_adj = mm(g1, g2, zero_diag=True)
        cur_x = dense_gcn(p_adj, xp, dWs[lvl], dbs[lvl], relu=True)
        perms.append(perm)
        if lvl < 2:
            xs.append(cur_x)
            adjs.append(p_adj)
            cur_src = p_adj

    up = jnp.zeros_like(xs[2]).at[perms[2]].set(cur_x)
    cur = xs[2] + up
    cur = dense_gcn(adjs[2], cur, uW0, ub0, relu=True)
    up = jnp.zeros_like(xs[1]).at[perms[1]].set(cur)
    cur = xs[1] + up
    cur = dense_gcn(adjs[1], cur, uW1, ub1, relu=True)
    up = jnp.zeros_like(xs[0]).at[perms[0]].set(cur)
    cur = xs[0] + up
    curp = jnp.zeros((np_, cur.shape[1]), jnp.float32).at[:n].set(cur)
    hs = scale_mm(curp, uW2, dinv0)
    z = mmt(adj, hs)
    out = gcn_epilogue(z, hs, dinv0, ub2, relu=False)[:n]

    return row_logsoftmax(out)


# speedup vs baseline: 2.7743x; 2.7743x over previous
import functools

import jax
import jax.numpy as jnp
from jax.experimental import pallas as pl

_KS = (2000, 1000, 500)


def _blk8(dim, pref):
    cands = [b for b in range(8, min(dim, pref) + 1, 8) if dim % b == 0]
    return max(cands) if cands else dim


def _blk128(dim, pref):
    cands = [b for b in range(128, min(dim, pref) + 1, 128) if dim % b == 0]
    return max(cands) if cands else dim


def _mm_kernel(a_ref, b_ref, ap_ref, dr_ref, dc_ref, o_ref, *, nc, bi, bj):
    c = pl.program_id(2)

    @pl.when(c == 0)
    def _():
        o_ref[...] = jnp.zeros_like(o_ref)

    o_ref[...] += jnp.dot(a_ref[...], b_ref[...],
                          preferred_element_type=jnp.float32)

    i = pl.program_id(0)
    j = pl.program_id(1)

    @pl.when(c == nc - 1)
    def _():
        corr = ap_ref[...].astype(jnp.float32) * (
            2.0 - dr_ref[...] - dc_ref[...])
        o = o_ref[...] + corr
        gi = i * bi + jax.lax.broadcasted_iota(jnp.int32, (bi, bj), 0)
        gj = j * bj + jax.lax.broadcasted_iota(jnp.int32, (bi, bj), 1)
        o_ref[...] = jnp.where(gi == gj, 0.0, o)


def mm_pool(a, b, a_pool, dp):
    m, k = a.shape
    k2, n = b.shape
    assert k == k2 and m == n
    bi = _blk8(m, 512)
    bj = _blk128(n, 2048)
    bc = _blk128(k, 2048)
    grid = (m // bi, n // bj, k // bc)
    return pl.pallas_call(
        functools.partial(_mm_kernel, nc=grid[2], bi=bi, bj=bj),
        grid=grid,
        in_specs=[
            pl.BlockSpec((bi, bc), lambda i, j, c: (i, c)),
            pl.BlockSpec((bc, bj), lambda i, j, c: (c, j)),
            pl.BlockSpec((bi, bj), lambda i, j, c: (i, j)),
            pl.BlockSpec((bi, 1), lambda i, j, c: (i, 0)),
            pl.BlockSpec((1, bj), lambda i, j, c: (0, j)),
        ],
        out_specs=pl.BlockSpec((bi, bj), lambda i, j, c: (i, j)),
        out_shape=jax.ShapeDtypeStruct((m, n), jnp.float32),
    )(a, b, a_pool, dp.reshape(m, 1), dp.reshape(1, m))


def _mmt_kernel(a_ref, h_ref, o_ref):
    c = pl.program_id(1)

    @pl.when(c == 0)
    def _():
        o_ref[...] = jnp.zeros_like(o_ref)

    o_ref[...] += jax.lax.dot_general(
        a_ref[...].astype(jnp.float32), h_ref[...], (((0,), (0,)), ((), ())),
        preferred_element_type=jnp.float32)


def mmt(a, h):
    s, d = a.shape
    s2, f = h.shape
    assert s == s2
    bd = _blk128(d, 1024)
    bs = _blk8(s, 2048)
    grid = (d // bd, s // bs)
    return pl.pallas_call(
        _mmt_kernel,
        grid=grid,
        in_specs=[
            pl.BlockSpec((bs, bd), lambda i, c: (c, i)),
            pl.BlockSpec((bs, f), lambda i, c: (c, 0)),
        ],
        out_specs=pl.BlockSpec((bd, f), lambda i, c: (i, 0)),
        out_shape=jax.ShapeDtypeStruct((d, f), jnp.float32),
    )(a, h)


def _scale_mm_kernel(x_ref, w_ref, s_ref, o_ref):
    o_ref[...] = s_ref[...] * jnp.dot(x_ref[...], w_ref[...],
                                      preferred_element_type=jnp.float32)


def scale_mm(x, w, s):
    n, f = x.shape
    bn = _blk8(n, 2048)
    grid = (n // bn,)
    return pl.pallas_call(
        _scale_mm_kernel,
        grid=grid,
        in_specs=[
            pl.BlockSpec((bn, f), lambda i: (i, 0)),
            pl.BlockSpec((f, w.shape[1]), lambda i: (0, 0)),
            pl.BlockSpec((bn, 1), lambda i: (i, 0)),
        ],
        out_specs=pl.BlockSpec((bn, w.shape[1]), lambda i: (i, 0)),
        out_shape=jax.ShapeDtypeStruct((n, w.shape[1]), jnp.float32),
    )(x, w, s)


def _epilogue_kernel(z_ref, hs_ref, s_ref, b_ref, o_ref, *, relu):
    o = s_ref[...] * (z_ref[...] + 2.0 * hs_ref[...]) + b_ref[...]
    if relu:
        o = jnp.maximum(o, 0.0)
    o_ref[...] = o


def gcn_epilogue(z, hs, s, b, relu):
    n, f = z.shape
    bn = _blk8(n, 2048)
    grid = (n // bn,)
    return pl.pallas_call(
        functools.partial(_epilogue_kernel, relu=relu),
        grid=grid,
        in_specs=[
            pl.BlockSpec((bn, f), lambda i: (i, 0)),
            pl.BlockSpec((bn, f), lambda i: (i, 0)),
            pl.BlockSpec((bn, 1), lambda i: (i, 0)),
            pl.BlockSpec((1, f), lambda i: (0, 0)),
        ],
        out_specs=pl.BlockSpec((bn, f), lambda i: (i, 0)),
        out_shape=jax.ShapeDtypeStruct((n, f), jnp.float32),
    )(z, hs, s, b.reshape(1, f))


def _dense_gcn_kernel(a_ref, x_ref, w_ref, b_ref, o_ref, *, relu):
    a = a_ref[...]
    h = jnp.dot(x_ref[...], w_ref[...], preferred_element_type=jnp.float32)
    deg = jnp.sum(a, axis=0) + 2.0
    dinv = jax.lax.rsqrt(deg)
    hs = dinv[:, None] * h
    z = jax.lax.dot_general(a, hs, (((0,), (0,)), ((), ())),
                            preferred_element_type=jnp.float32)
    o = dinv[:, None] * (z + 2.0 * hs) + b_ref[...]
    if relu:
        o = jnp.maximum(o, 0.0)
    o_ref[...] = o


def dense_gcn(a, x, w, b, relu):
    k, f = x.shape
    return pl.pallas_call(
        functools.partial(_dense_gcn_kernel, relu=relu),
        in_specs=[
            pl.BlockSpec(a.shape, lambda: (0, 0)),
            pl.BlockSpec((k, f), lambda: (0, 0)),
            pl.BlockSpec(w.shape, lambda: (0, 0)),
            pl.BlockSpec((1, w.shape[1]), lambda: (0, 0)),
        ],
        out_specs=pl.BlockSpec((k, w.shape[1]), lambda: (0, 0)),
        out_shape=jax.ShapeDtypeStruct((k, w.shape[1]), jnp.float32),
    )(a, x, w, b.reshape(1, -1))


def _logsoftmax_kernel(x_ref, o_ref):
    x = x_ref[...]
    m = jnp.max(x, axis=1, keepdims=True)
    lse = m + jnp.log(jnp.sum(jnp.exp(x - m), axis=1, keepdims=True))
    o_ref[...] = x - lse


def row_logsoftmax(x):
    n, f = x.shape
    bn = _blk8(n, 2048)
    return pl.pallas_call(
        _logsoftmax_kernel,
        grid=(n // bn,),
        in_specs=[pl.BlockSpec((bn, f), lambda i: (i, 0))],
        out_specs=pl.BlockSpec((bn, f), lambda i: (i, 0)),
        out_shape=jax.ShapeDtypeStruct((n, f), jnp.float32),
    )(x)


def kernel(x, edge_index, dW0, dW1, dW2, dW3, db0, db1, db2, db3,
           pw0, pw1, pw2, uW0, uW1, uW2, ub0, ub1, ub2):
    n = x.shape[0]
    np_ = ((n + 127) // 128) * 128
    if np_ % 1024:
        np_ = ((n + 1023) // 1024) * 1024
    adj = jnp.zeros((np_, np_), jnp.bfloat16).at[
        edge_index[0], edge_index[1]].add(1.0)

    xpad = jnp.zeros((np_, x.shape[1]), jnp.float32).at[:n].set(x)

    deg0 = jnp.sum(adj, axis=0, dtype=jnp.float32) + 2.0
    dinv0 = jax.lax.rsqrt(deg0).reshape(np_, 1)
    hs = scale_mm(xpad, dW0, dinv0)
    z = mmt(adj, hs)
    x1 = gcn_epilogue(z, hs, dinv0, db0, relu=True)[:n]

    pws = (pw0, pw1, pw2)
    dWs = (dW1, dW2, dW3)
    dbs = (db1, db2, db3)

    xs = [x1]
    adjs = [None]
    perms = []

    cur_x = x1
    cur_src = adj
    for lvl in range(3):
        k = _KS[lvl]
        w = pws[lvl]
        score = cur_x @ (w / jnp.linalg.norm(w))
        top_vals, perm = jax.lax.top_k(score, k)
        xp = cur_x[perm] * jnp.tanh(top_vals)[:, None]
        g1 = cur_src[perm, :]
        g2 = cur_src[:, perm]
        a_pool = g1[:, perm]
        dp = jnp.diagonal(a_pool).astype(jnp.float32)
        p_adj = mm_pool(g1, g2, a_pool, dp)
        cur_x = dense_gcn(p_adj, xp, dWs[lvl], dbs[lvl], relu=True)
        perms.append(perm)
        if lvl < 2:
            xs.append(cur_x)
            adjs.append(p_adj)
            cur_src = p_adj

    up = jnp.zeros_like(xs[2]).at[perms[2]].set(cur_x)
    cur = xs[2] + up
    cur = dense_gcn(adjs[2], cur, uW0, ub0, relu=True)
    up = jnp.zeros_like(xs[1]).at[perms[1]].set(cur)
    cur = xs[1] + up
    cur = dense_gcn(adjs[1], cur, uW1, ub1, relu=True)
    up = jnp.zeros_like(xs[0]).at[perms[0]].set(cur)
    cur = xs[0] + up
    curp = jnp.zeros((np_, cur.shape[1]), jnp.float32).at[:n].set(cur)
    hs = scale_mm(curp, uW2, dinv0)
    z = mmt(adj, hs)
    out = gcn_epilogue(z, hs, dinv0, ub2, relu=False)[:n]

    return row_logsoftmax(out)
